# baseline (speedup 1.0000x reference)
import jax
import jax.numpy as jnp
from jax.experimental import pallas as pl


def _body(nodes_ref, pooled_ref, maskf_ref, lockedf_ref, out_ref):
    keep = 1.0 - maskf_ref[...] * lockedf_ref[...]
    out_ref[...] = nodes_ref[...] + pooled_ref[...] * keep


def kernel(nodes, mask, pooled_edges, locked_nodes):
    B, N, D = nodes.shape
    maskf = mask.astype(jnp.float32).reshape(B, N, 1)
    lockedf = locked_nodes.astype(jnp.float32).reshape(B, N, 1)
    Bb = 10
    bs3 = pl.BlockSpec((Bb, N, D), lambda i: (i, 0, 0))
    bsm = pl.BlockSpec((Bb, N, 1), lambda i: (i, 0, 0))
    return pl.pallas_call(
        _body,
        grid=(B // Bb,),
        in_specs=[bs3, bs3, bsm, bsm],
        out_specs=bs3,
        out_shape=jax.ShapeDtypeStruct((B, N, D), nodes.dtype),
    )(nodes, pooled_edges, maskf, lockedf)


# speedup vs baseline: 2.2849x; 2.2849x over previous
import jax
import jax.numpy as jnp
from jax.experimental import pallas as pl

_BB = 10


def _body(nodes_ref, pooled_ref, maskf_ref, lockedf_ref, out_ref):
    m = maskf_ref[0] * lockedf_ref[0]
    keep = (1.0 - m)[:, :, None]
    out_ref[...] = nodes_ref[...] + pooled_ref[...] * keep


def kernel(nodes, mask, pooled_edges, locked_nodes):
    B, N, D = nodes.shape
    nb = B // _BB
    maskf = mask.astype(jnp.float32).reshape(nb, _BB, N)
    lockedf = locked_nodes.astype(jnp.float32).reshape(nb, _BB, N)
    bs3 = pl.BlockSpec((_BB, N, D), lambda i: (i, 0, 0))
    bsm = pl.BlockSpec((1, _BB, N), lambda i: (i, 0, 0))
    return pl.pallas_call(
        _body,
        grid=(nb,),
        in_specs=[bs3, bs3, bsm, bsm],
        out_specs=bs3,
        out_shape=jax.ShapeDtypeStruct((B, N, D), nodes.dtype),
    )(nodes, pooled_edges, maskf, lockedf)
